# Initial kernel scaffold; baseline (speedup 1.0000x reference)
import jax
import jax.numpy as jnp
from jax.experimental import pallas as pl


def kernel(x, edge_index, W_lin0, b_lin0, W_conv0, b_conv0, W_out, b_out):
    raise NotImplementedError("write your pallas kernel here")


# speedup vs baseline: 7.9455x; 7.9455x over previous
import functools

import jax
import jax.numpy as jnp
from jax import lax
from jax.experimental import pallas as pl
from jax.experimental.pallas import tpu as pltpu
from jax.experimental.pallas import tpu_sc as plsc

NC = 2
NS = 16
NW = NC * NS
L = 16
CHUNK = 128


def _sc_mesh():
    return plsc.VectorSubcoreMesh(
        core_axis_name="c", subcore_axis_name="s", num_cores=NC,
        num_subcores=NS)


@functools.partial(jax.jit, static_argnums=(2, 3))
def _sc_degrees(src, dst, E, NB):
    EP = E // NW

    @functools.partial(
        pl.kernel,
        out_type=jax.ShapeDtypeStruct((2 * NW, NB, 128), jnp.float32),
        mesh=_sc_mesh(),
        compiler_params=pltpu.CompilerParams(needs_layout_passes=False),
        scratch_types=[
            pltpu.VMEM((EP,), jnp.int32),
            pltpu.VMEM((EP,), jnp.int32),
            pltpu.VMEM((NB, 128), jnp.float32),
            pltpu.VMEM((NB, 128), jnp.float32),
        ],
    )
    def deg_kernel(src_hbm, dst_hbm, out_hbm, sbuf, dbuf, hsrc, hdst):
        cid = lax.axis_index("c")
        sid = lax.axis_index("s")
        wid = sid * NC + cid
        base = wid * EP

        zeros = jnp.zeros((L,), jnp.float32)

        def zero_body(i, _):
            r = i // 8
            j = i % 8
            hsrc[r, pl.ds(j * L, L)] = zeros
            hdst[r, pl.ds(j * L, L)] = zeros
            return 0

        lax.fori_loop(0, NB * 8, zero_body, 0)

        pltpu.sync_copy(src_hbm.at[pl.ds(base, EP)], sbuf)
        pltpu.sync_copy(dst_hbm.at[pl.ds(base, EP)], dbuf)

        ones = jnp.ones((L,), jnp.float32)
        seven = jnp.int32(7)
        mask = jnp.int32(127)

        def hist_body(i, _):
            s = sbuf[pl.ds(i * L, L)]
            d = dbuf[pl.ds(i * L, L)]
            plsc.addupdate_scatter(
                hsrc, [lax.shift_right_logical(s, seven), s & mask], ones)
            plsc.addupdate_scatter(
                hdst, [lax.shift_right_logical(d, seven), d & mask], ones)
            return 0

        lax.fori_loop(0, EP // L, hist_body, 0)

        pltpu.sync_copy(hsrc, out_hbm.at[wid])
        pltpu.sync_copy(hdst, out_hbm.at[NW + wid])

    return deg_kernel(src, dst)


def _col_scale(mat_n2_d, norm_nb_128, NB, D):
    nb = jnp.broadcast_to(norm_nb_128[:, None, :], (NB, 128, 128))
    nT = jnp.transpose(nb, (0, 2, 1))
    m3 = mat_n2_d.reshape(NB, 128, D)
    return (m3 * nT).reshape(NB * 128, D)


def _tc_dense(xp, W1, b1, W2, degp, NB):
    N2, D = xp.shape

    def body(x_ref, w1_ref, b1_ref, w2_ref, degp_ref, hw_ref, ndst_ref):
        dp = degp_ref[...]
        deg_src = jnp.sum(dp[:NW], axis=0)
        deg_dst = jnp.sum(dp[NW:], axis=0)
        nsrc = jnp.where(deg_src > 0.0,
                         lax.rsqrt(jnp.maximum(deg_src, 1.0)), 0.0)
        ndst = jnp.where(deg_dst > 0.0,
                         lax.rsqrt(jnp.maximum(deg_dst, 1.0)), 0.0)
        ndst_ref[...] = ndst
        h = jnp.dot(x_ref[...], w1_ref[...],
                    preferred_element_type=jnp.float32)
        h = jnp.maximum(h + b1_ref[...][None, :], 0.0)
        hw = jnp.dot(h, w2_ref[...], preferred_element_type=jnp.float32)
        hw_ref[...] = _col_scale(hw, nsrc, NB, D)

    return pl.pallas_call(
        body,
        out_shape=(
            jax.ShapeDtypeStruct((N2, D), jnp.float32),
            jax.ShapeDtypeStruct((NB, 128), jnp.float32),
        ),
    )(xp, W1, b1, W2, degp)


@functools.partial(jax.jit, static_argnums=(3, 4, 5))
def _sc_scatter(srcp, dstp, hw, EP2, N, R):
    D = hw.shape[1]
    n_chunks = EP2 // CHUNK
    rows_per_tile = R // NS
    zb = 128
    n_zero = rows_per_tile // zb

    @functools.partial(
        pl.kernel,
        out_type=jax.ShapeDtypeStruct((2 * R, D), jnp.float32),
        mesh=_sc_mesh(),
        scratch_types=[
            pltpu.VMEM((CHUNK,), jnp.int32),
            pltpu.VMEM((CHUNK,), jnp.int32),
            pltpu.VMEM((CHUNK, D), jnp.float32),
            pltpu.VMEM_SHARED((R, D), jnp.float32),
            pltpu.SemaphoreType.DMA,
        ],
    )
    def scat_kernel(src_hbm, dst_hbm, hw_hbm, out_hbm, sbuf, dbuf, rows,
                    acc_sh, sem):
        cid = lax.axis_index("c")
        sid = lax.axis_index("s")
        wid = sid * NC + cid

        zeros = jnp.zeros((L,), jnp.float32)

        def zrow(i, _):
            r = i // (D // L)
            j = i % (D // L)
            rows[r, pl.ds(j * L, L)] = zeros
            return 0

        lax.fori_loop(0, zb * (D // L), zrow, 0)

        for b in range(n_zero):
            pltpu.sync_copy(
                rows.at[pl.ds(0, zb)],
                acc_sh.at[pl.ds(sid * rows_per_tile + b * zb, zb)])
        plsc.subcore_barrier()

        def chunk_body(c, _):
            base = wid * EP2 + c * CHUNK
            pltpu.sync_copy(src_hbm.at[pl.ds(base, CHUNK)], sbuf)
            pltpu.sync_copy(dst_hbm.at[pl.ds(base, CHUNK)], dbuf)
            pltpu.async_copy(hw_hbm.at[sbuf], rows, sem).wait()
            pltpu.sync_copy(rows, acc_sh.at[dbuf], add=True)
            return 0

        lax.fori_loop(0, n_chunks, chunk_body, 0)
        plsc.subcore_barrier()

        pltpu.sync_copy(
            acc_sh.at[pl.ds(sid * rows_per_tile, rows_per_tile)],
            out_hbm.at[pl.ds(cid * R + sid * rows_per_tile, rows_per_tile)])

    return scat_kernel(srcp, dstp, hw)


def _tc_final(accp, ndst, b2, W3, b3, NB, R):
    D = accp.shape[1]
    DO = W3.shape[1]
    N2 = NB * 128

    def body(accp_ref, ndst_ref, b2_ref, w3_ref, b3_ref, out_ref):
        a = accp_ref[:N2, :] + accp_ref[R:R + N2, :]
        h2 = _col_scale(a, ndst_ref[...], NB, D) + b2_ref[...][None, :]
        h2 = jnp.maximum(h2, 0.0)
        out_ref[...] = (jnp.dot(h2, w3_ref[...],
                                preferred_element_type=jnp.float32)
                        + b3_ref[...][None, :])

    return pl.pallas_call(
        body,
        out_shape=jax.ShapeDtypeStruct((N2, DO), jnp.float32),
    )(accp, ndst, b2, W3, b3)


def kernel(x, edge_index, W_lin0, b_lin0, W_conv0, b_conv0, W_out, b_out):
    N, D = x.shape
    E = edge_index.shape[1]
    NB = -(-N // 128)
    N2 = NB * 128

    src = edge_index[0]
    dst = edge_index[1]

    degp = _sc_degrees(src, dst, E, NB)
    xp = jnp.pad(x, ((0, N2 - N), (0, 0))) if N2 > N else x
    hw, ndst = _tc_dense(xp, W_lin0, b_lin0, W_conv0, degp, NB)

    EP2 = -(-E // (NW * CHUNK)) * CHUNK
    pad = EP2 * NW - E
    R = -(-N // (NS * 128)) * (NS * 128)
    if pad > 0 and R == N:
        R += NS * 128
    if pad > 0:
        n_dummy = R - N
        j = jnp.arange(pad, dtype=jnp.int32)
        pad_src = (j * 641) % jnp.int32(N)
        pad_dst = jnp.int32(N) + (j % jnp.int32(n_dummy))
        srcp = jnp.concatenate([src, pad_src])
        dstp = jnp.concatenate([dst, pad_dst])
    else:
        srcp, dstp = src, dst

    accp = _sc_scatter(srcp, dstp, hw, EP2, N, R)
    out = _tc_final(accp, ndst, b_conv0, W_out, b_out, NB, R)
    return out[:N]


# speedup vs baseline: 11.4719x; 1.4438x over previous
import functools

import jax
import jax.numpy as jnp
from jax import lax
from jax.experimental import pallas as pl
from jax.experimental.pallas import tpu as pltpu
from jax.experimental.pallas import tpu_sc as plsc

NC = 2
NS = 16
NW = NC * NS
L = 16
CHUNK = 128


def _sc_mesh():
    return plsc.VectorSubcoreMesh(
        core_axis_name="c", subcore_axis_name="s", num_cores=NC,
        num_subcores=NS)


@functools.partial(jax.jit, static_argnums=(2, 3))
def _sc_degrees(src, dst, E, NB):
    EP = E // NW

    @functools.partial(
        pl.kernel,
        out_type=jax.ShapeDtypeStruct((2 * NW, NB, 128), jnp.float32),
        mesh=_sc_mesh(),
        compiler_params=pltpu.CompilerParams(needs_layout_passes=False),
        scratch_types=[
            pltpu.VMEM((EP,), jnp.int32),
            pltpu.VMEM((EP,), jnp.int32),
            pltpu.VMEM((NB, 128), jnp.float32),
            pltpu.VMEM((NB, 128), jnp.float32),
        ],
    )
    def deg_kernel(src_hbm, dst_hbm, out_hbm, sbuf, dbuf, hsrc, hdst):
        cid = lax.axis_index("c")
        sid = lax.axis_index("s")
        wid = sid * NC + cid
        base = wid * EP

        zeros = jnp.zeros((L,), jnp.float32)

        def zero_body(i, _):
            r = i // 8
            j = i % 8
            hsrc[r, pl.ds(j * L, L)] = zeros
            hdst[r, pl.ds(j * L, L)] = zeros
            return 0

        lax.fori_loop(0, NB * 8, zero_body, 0)

        pltpu.sync_copy(src_hbm.at[pl.ds(base, EP)], sbuf)
        pltpu.sync_copy(dst_hbm.at[pl.ds(base, EP)], dbuf)

        ones = jnp.ones((L,), jnp.float32)
        seven = jnp.int32(7)
        mask = jnp.int32(127)

        def hist_body(i, _):
            s = sbuf[pl.ds(i * L, L)]
            d = dbuf[pl.ds(i * L, L)]
            plsc.addupdate_scatter(
                hsrc, [lax.shift_right_logical(s, seven), s & mask], ones)
            plsc.addupdate_scatter(
                hdst, [lax.shift_right_logical(d, seven), d & mask], ones)
            return 0

        lax.fori_loop(0, EP // L, hist_body, 0)

        pltpu.sync_copy(hsrc, out_hbm.at[wid])
        pltpu.sync_copy(hdst, out_hbm.at[NW + wid])

    return deg_kernel(src, dst)


def _col_scale(mat_n2_d, norm_nb_128, NB, D):
    nb = jnp.broadcast_to(norm_nb_128[:, None, :], (NB, 128, 128))
    nT = jnp.transpose(nb, (0, 2, 1))
    m3 = mat_n2_d.reshape(NB, 128, D)
    return (m3 * nT).reshape(NB * 128, D)


def _tc_dense(xp, W1, b1, W2, degp, NB):
    N2, D = xp.shape

    def body(x_ref, w1_ref, b1_ref, w2_ref, degp_ref, hw_ref, ndst_ref):
        dp = degp_ref[...]
        deg_src = jnp.sum(dp[:NW], axis=0)
        deg_dst = jnp.sum(dp[NW:], axis=0)
        nsrc = jnp.where(deg_src > 0.0,
                         lax.rsqrt(jnp.maximum(deg_src, 1.0)), 0.0)
        ndst = jnp.where(deg_dst > 0.0,
                         lax.rsqrt(jnp.maximum(deg_dst, 1.0)), 0.0)
        ndst_ref[...] = ndst
        h = jnp.dot(x_ref[...], w1_ref[...],
                    preferred_element_type=jnp.float32)
        h = jnp.maximum(h + b1_ref[...][None, :], 0.0)
        hw = jnp.dot(h, w2_ref[...], preferred_element_type=jnp.float32)
        hw_ref[...] = _col_scale(hw, nsrc, NB, D)

    return pl.pallas_call(
        body,
        out_shape=(
            jax.ShapeDtypeStruct((N2, D), jnp.float32),
            jax.ShapeDtypeStruct((NB, 128), jnp.float32),
        ),
    )(xp, W1, b1, W2, degp)


@functools.partial(jax.jit, static_argnums=(3, 4, 5))
def _sc_scatter(srcp, dstp, hw, EP2, N, R):
    D = hw.shape[1]
    n_chunks = EP2 // CHUNK
    assert n_chunks % 2 == 0
    rows_per_tile = R // NS
    zb = 64
    n_zero = rows_per_tile // zb

    @functools.partial(
        pl.kernel,
        out_type=jax.ShapeDtypeStruct((2 * R, D), jnp.float32),
        mesh=_sc_mesh(),
        scratch_types=[
            pltpu.VMEM((2, CHUNK), jnp.int32),
            pltpu.VMEM((2, CHUNK), jnp.int32),
            pltpu.VMEM((CHUNK, D), jnp.float32),
            pltpu.VMEM((CHUNK, D), jnp.float32),
            pltpu.VMEM_SHARED((R, D), jnp.float32),
            pltpu.SemaphoreType.DMA,
            pltpu.SemaphoreType.DMA,
            pltpu.SemaphoreType.DMA,
            pltpu.SemaphoreType.DMA,
        ],
    )
    def scat_kernel(src_hbm, dst_hbm, hw_hbm, out_hbm, ixA, ixB, rowsA,
                    rowsB, acc_sh, gsA, gsB, ssA, ssB):
        cid = lax.axis_index("c")
        sid = lax.axis_index("s")
        wid = sid * NC + cid
        ebase = wid * EP2

        def load_idx(ix, c):
            pltpu.sync_copy(src_hbm.at[pl.ds(ebase + c * CHUNK, CHUNK)],
                            ix.at[0])
            pltpu.sync_copy(dst_hbm.at[pl.ds(ebase + c * CHUNK, CHUNK)],
                            ix.at[1])

        zeros = jnp.zeros((L,), jnp.float32)

        def zrow(i, _):
            r = i // (D // L)
            j = i % (D // L)
            rowsA[r, pl.ds(j * L, L)] = zeros
            return 0

        lax.fori_loop(0, zb * (D // L), zrow, 0)

        for b in range(n_zero):
            pltpu.sync_copy(
                rowsA.at[pl.ds(0, zb)],
                acc_sh.at[pl.ds(sid * rows_per_tile + b * zb, zb)])
        plsc.subcore_barrier()

        load_idx(ixA, 0)
        pltpu.async_copy(hw_hbm.at[ixA.at[0]], rowsA, gsA)
        load_idx(ixB, 1)
        pltpu.async_copy(hw_hbm.at[ixB.at[0]], rowsB, gsB)

        def chunk_body(c, _):
            pltpu.make_async_copy(hw_hbm.at[ixA.at[0]], rowsA, gsA).wait()
            pltpu.async_copy(rowsA, acc_sh.at[ixA.at[1]], ssA, add=True)
            pltpu.make_async_copy(hw_hbm.at[ixB.at[0]], rowsB, gsB).wait()
            pltpu.async_copy(rowsB, acc_sh.at[ixB.at[1]], ssB, add=True)

            @pl.when(c + 2 < n_chunks)
            def _():
                pltpu.make_async_copy(rowsA, acc_sh.at[ixA.at[1]],
                                      ssA).wait()
                load_idx(ixA, c + 2)
                pltpu.async_copy(hw_hbm.at[ixA.at[0]], rowsA, gsA)
                pltpu.make_async_copy(rowsB, acc_sh.at[ixB.at[1]],
                                      ssB).wait()
                load_idx(ixB, c + 3)
                pltpu.async_copy(hw_hbm.at[ixB.at[0]], rowsB, gsB)

            @pl.when(c + 2 >= n_chunks)
            def _():
                pltpu.make_async_copy(rowsA, acc_sh.at[ixA.at[1]],
                                      ssA).wait()
                pltpu.make_async_copy(rowsB, acc_sh.at[ixB.at[1]],
                                      ssB).wait()
            return 0

        lax.fori_loop(0, n_chunks // 2, lambda i, _:
                      chunk_body(i * 2, _), 0)
        plsc.subcore_barrier()

        pltpu.sync_copy(
            acc_sh.at[pl.ds(sid * rows_per_tile, rows_per_tile)],
            out_hbm.at[pl.ds(cid * R + sid * rows_per_tile, rows_per_tile)])

    return scat_kernel(srcp, dstp, hw)


def _tc_final(accp, ndst, b2, W3, b3, NB, R):
    D = accp.shape[1]
    DO = W3.shape[1]
    N2 = NB * 128

    def body(accp_ref, ndst_ref, b2_ref, w3_ref, b3_ref, out_ref):
        a = accp_ref[:N2, :] + accp_ref[R:R + N2, :]
        h2 = _col_scale(a, ndst_ref[...], NB, D) + b2_ref[...][None, :]
        h2 = jnp.maximum(h2, 0.0)
        out_ref[...] = (jnp.dot(h2, w3_ref[...],
                                preferred_element_type=jnp.float32)
                        + b3_ref[...][None, :])

    return pl.pallas_call(
        body,
        out_shape=jax.ShapeDtypeStruct((N2, DO), jnp.float32),
    )(accp, ndst, b2, W3, b3)


def kernel(x, edge_index, W_lin0, b_lin0, W_conv0, b_conv0, W_out, b_out):
    N, D = x.shape
    E = edge_index.shape[1]
    NB = -(-N // 128)
    N2 = NB * 128

    src = edge_index[0]
    dst = edge_index[1]

    degp = _sc_degrees(src, dst, E, NB)
    xp = jnp.pad(x, ((0, N2 - N), (0, 0))) if N2 > N else x
    hw, ndst = _tc_dense(xp, W_lin0, b_lin0, W_conv0, degp, NB)

    n_chunks = -(-(-(-E // (NW * CHUNK))) // 8) * 8
    EP2 = n_chunks * CHUNK
    pad = EP2 * NW - E
    R = -(-N // (NS * 128)) * (NS * 128)
    if pad > 0 and R == N:
        R += NS * 128
    if pad > 0:
        n_dummy = R - N
        j = jnp.arange(pad, dtype=jnp.int32)
        pad_src = (j * 641) % jnp.int32(N)
        pad_dst = jnp.int32(N) + (j % jnp.int32(n_dummy))
        srcp = jnp.concatenate([src, pad_src])
        dstp = jnp.concatenate([dst, pad_dst])
    else:
        srcp, dstp = src, dst
    accp = _sc_scatter(srcp, dstp, hw, EP2, N, R)
    out = _tc_final(accp, ndst, b_conv0, W_out, b_out, NB, R)
    return out[:N]


# speedup vs baseline: 11.8136x; 1.0298x over previous
import functools

import jax
import jax.numpy as jnp
from jax import lax
from jax.experimental import pallas as pl
from jax.experimental.pallas import tpu as pltpu
from jax.experimental.pallas import tpu_sc as plsc

NC = 2
NS = 16
NW = NC * NS
L = 16
CHUNK = 128


def _sc_mesh():
    return plsc.VectorSubcoreMesh(
        core_axis_name="c", subcore_axis_name="s", num_cores=NC,
        num_subcores=NS)


@functools.partial(jax.jit, static_argnums=(2, 3))
def _sc_degrees(src, dst, E, NB):
    EP = E // NW

    @functools.partial(
        pl.kernel,
        out_type=jax.ShapeDtypeStruct((2 * NW, NB, 128), jnp.float32),
        mesh=_sc_mesh(),
        compiler_params=pltpu.CompilerParams(needs_layout_passes=False),
        scratch_types=[
            pltpu.VMEM((EP,), jnp.int32),
            pltpu.VMEM((EP,), jnp.int32),
            pltpu.VMEM((NB, 128), jnp.float32),
            pltpu.VMEM((NB, 128), jnp.float32),
        ],
    )
    def deg_kernel(src_hbm, dst_hbm, out_hbm, sbuf, dbuf, hsrc, hdst):
        cid = lax.axis_index("c")
        sid = lax.axis_index("s")
        wid = sid * NC + cid
        base = wid * EP

        zeros = jnp.zeros((L,), jnp.float32)

        def zero_body(i, _):
            r = i // 8
            j = i % 8
            hsrc[r, pl.ds(j * L, L)] = zeros
            hdst[r, pl.ds(j * L, L)] = zeros
            return 0

        lax.fori_loop(0, NB * 8, zero_body, 0)

        pltpu.sync_copy(src_hbm.at[pl.ds(base, EP)], sbuf)
        pltpu.sync_copy(dst_hbm.at[pl.ds(base, EP)], dbuf)

        ones = jnp.ones((L,), jnp.float32)
        seven = jnp.int32(7)
        mask = jnp.int32(127)

        def hist_body(i, _):
            s = sbuf[pl.ds(i * L, L)]
            d = dbuf[pl.ds(i * L, L)]
            plsc.addupdate_scatter(
                hsrc, [lax.shift_right_logical(s, seven), s & mask], ones)
            plsc.addupdate_scatter(
                hdst, [lax.shift_right_logical(d, seven), d & mask], ones)
            return 0

        lax.fori_loop(0, EP // L, hist_body, 0)

        pltpu.sync_copy(hsrc, out_hbm.at[wid])
        pltpu.sync_copy(hdst, out_hbm.at[NW + wid])

    return deg_kernel(src, dst)


def _col_scale(mat_n2_d, norm_nb_128, NB, D):
    nb = jnp.broadcast_to(norm_nb_128[:, None, :], (NB, 128, 128))
    nT = jnp.transpose(nb, (0, 2, 1))
    m3 = mat_n2_d.reshape(NB, 128, D)
    return (m3 * nT).reshape(NB * 128, D)


def _tc_dense(xp, W1, b1, W2, degp, NB):
    N2, D = xp.shape

    def body(x_ref, w1_ref, b1_ref, w2_ref, degp_ref, hw_ref, ndst_ref):
        dp = degp_ref[...]
        deg_src = jnp.sum(dp[:NW], axis=0)
        deg_dst = jnp.sum(dp[NW:], axis=0)
        nsrc = jnp.where(deg_src > 0.0,
                         lax.rsqrt(jnp.maximum(deg_src, 1.0)), 0.0)
        ndst = jnp.where(deg_dst > 0.0,
                         lax.rsqrt(jnp.maximum(deg_dst, 1.0)), 0.0)
        ndst_ref[...] = ndst
        h = jnp.dot(x_ref[...], w1_ref[...],
                    preferred_element_type=jnp.float32)
        h = jnp.maximum(h + b1_ref[...][None, :], 0.0)
        hw = jnp.dot(h, w2_ref[...], preferred_element_type=jnp.float32)
        hw_ref[...] = _col_scale(hw, nsrc, NB, D)

    return pl.pallas_call(
        body,
        out_shape=(
            jax.ShapeDtypeStruct((N2, D), jnp.float32),
            jax.ShapeDtypeStruct((NB, 128), jnp.float32),
        ),
    )(xp, W1, b1, W2, degp)


@functools.partial(jax.jit, static_argnums=(3, 4, 5))
def _sc_scatter(srcp, dstp, hw, EP2, N, R):
    D = hw.shape[1]
    n_chunks = EP2 // CHUNK
    assert n_chunks % 2 == 0
    rows_per_tile = R // NS
    zb = 64
    n_zero = rows_per_tile // zb

    @functools.partial(
        pl.kernel,
        out_type=jax.ShapeDtypeStruct((2 * R, D), jnp.float32),
        mesh=_sc_mesh(),
        scratch_types=[
            pltpu.VMEM((2, CHUNK), jnp.int32),
            pltpu.VMEM((2, CHUNK), jnp.int32),
            pltpu.VMEM((2, CHUNK), jnp.int32),
            pltpu.VMEM((2, CHUNK), jnp.int32),
            pltpu.VMEM((CHUNK, D), jnp.float32),
            pltpu.VMEM((CHUNK, D), jnp.float32),
            pltpu.VMEM_SHARED((R, D), jnp.float32),
            pltpu.SemaphoreType.DMA,
            pltpu.SemaphoreType.DMA,
            pltpu.SemaphoreType.DMA,
            pltpu.SemaphoreType.DMA,
            pltpu.SemaphoreType.DMA,
            pltpu.SemaphoreType.DMA,
            pltpu.SemaphoreType.DMA,
            pltpu.SemaphoreType.DMA,
        ],
    )
    def scat_kernel(src_hbm, dst_hbm, hw_hbm, out_hbm, ixA0, ixA1, ixB0,
                    ixB1, rowsA, rowsB, acc_sh, gsA, gsB, ssA, ssB,
                    isA0, isA1, isB0, isB1):
        cid = lax.axis_index("c")
        sid = lax.axis_index("s")
        wid = sid * NC + cid
        ebase = wid * EP2

        def load_idx(ix, sem, c):
            pltpu.async_copy(src_hbm.at[pl.ds(ebase + c * CHUNK, CHUNK)],
                             ix.at[0], sem)
            pltpu.async_copy(dst_hbm.at[pl.ds(ebase + c * CHUNK, CHUNK)],
                             ix.at[1], sem)

        def wait_idx(ix, sem):
            pltpu.make_async_copy(src_hbm.at[pl.ds(0, CHUNK)], ix.at[0],
                                  sem).wait()
            pltpu.make_async_copy(src_hbm.at[pl.ds(0, CHUNK)], ix.at[1],
                                  sem).wait()

        zeros = jnp.zeros((L,), jnp.float32)

        def zrow(i, _):
            r = i // (D // L)
            j = i % (D // L)
            rowsA[r, pl.ds(j * L, L)] = zeros
            return 0

        lax.fori_loop(0, zb * (D // L), zrow, 0)

        for b in range(n_zero):
            pltpu.sync_copy(
                rowsA.at[pl.ds(0, zb)],
                acc_sh.at[pl.ds(sid * rows_per_tile + b * zb, zb)])
        plsc.subcore_barrier()

        load_idx(ixA0, isA0, 0)
        load_idx(ixB0, isB0, 1)
        load_idx(ixA1, isA1, 2)
        load_idx(ixB1, isB1, 3)
        wait_idx(ixA0, isA0)
        pltpu.async_copy(hw_hbm.at[ixA0.at[0]], rowsA, gsA)
        wait_idx(ixB0, isB0)
        pltpu.async_copy(hw_hbm.at[ixB0.at[0]], rowsB, gsB)

        def half(c, ix_cur_A, is_cur_A, ix_nxt_A, is_nxt_A,
                 ix_cur_B, is_cur_B, ix_nxt_B, is_nxt_B):
            pltpu.make_async_copy(hw_hbm.at[ix_cur_A.at[0]], rowsA,
                                  gsA).wait()
            pltpu.async_copy(rowsA, acc_sh.at[ix_cur_A.at[1]], ssA,
                             add=True)
            pltpu.make_async_copy(hw_hbm.at[ix_cur_B.at[0]], rowsB,
                                  gsB).wait()
            pltpu.async_copy(rowsB, acc_sh.at[ix_cur_B.at[1]], ssB,
                             add=True)

            pltpu.make_async_copy(rowsA, acc_sh.at[ix_cur_A.at[1]],
                                  ssA).wait()

            @pl.when(c + 2 < n_chunks)
            def _():
                wait_idx(ix_nxt_A, is_nxt_A)
                pltpu.async_copy(hw_hbm.at[ix_nxt_A.at[0]], rowsA, gsA)

            @pl.when(c + 4 < n_chunks)
            def _():
                load_idx(ix_cur_A, is_cur_A, c + 4)

            pltpu.make_async_copy(rowsB, acc_sh.at[ix_cur_B.at[1]],
                                  ssB).wait()

            @pl.when(c + 3 < n_chunks)
            def _():
                wait_idx(ix_nxt_B, is_nxt_B)
                pltpu.async_copy(hw_hbm.at[ix_nxt_B.at[0]], rowsB, gsB)

            @pl.when(c + 5 < n_chunks)
            def _():
                load_idx(ix_cur_B, is_cur_B, c + 5)

            return 0

        def quad_body(i, _):
            c = i * 4
            half(c, ixA0, isA0, ixA1, isA1, ixB0, isB0, ixB1, isB1)
            half(c + 2, ixA1, isA1, ixA0, isA0, ixB1, isB1, ixB0, isB0)
            return 0

        lax.fori_loop(0, n_chunks // 4, quad_body, 0)
        plsc.subcore_barrier()

        pltpu.sync_copy(
            acc_sh.at[pl.ds(sid * rows_per_tile, rows_per_tile)],
            out_hbm.at[pl.ds(cid * R + sid * rows_per_tile, rows_per_tile)])

    return scat_kernel(srcp, dstp, hw)


def _tc_final(accp, ndst, b2, W3, b3, NB, R):
    D = accp.shape[1]
    DO = W3.shape[1]
    N2 = NB * 128

    def body(accp_ref, ndst_ref, b2_ref, w3_ref, b3_ref, out_ref):
        a = accp_ref[:N2, :] + accp_ref[R:R + N2, :]
        h2 = _col_scale(a, ndst_ref[...], NB, D) + b2_ref[...][None, :]
        h2 = jnp.maximum(h2, 0.0)
        out_ref[...] = (jnp.dot(h2, w3_ref[...],
                                preferred_element_type=jnp.float32)
                        + b3_ref[...][None, :])

    return pl.pallas_call(
        body,
        out_shape=jax.ShapeDtypeStruct((N2, DO), jnp.float32),
    )(accp, ndst, b2, W3, b3)


def kernel(x, edge_index, W_lin0, b_lin0, W_conv0, b_conv0, W_out, b_out):
    N, D = x.shape
    E = edge_index.shape[1]
    NB = -(-N // 128)
    N2 = NB * 128

    src = edge_index[0]
    dst = edge_index[1]

    degp = _sc_degrees(src, dst, E, NB)
    xp = jnp.pad(x, ((0, N2 - N), (0, 0))) if N2 > N else x
    hw, ndst = _tc_dense(xp, W_lin0, b_lin0, W_conv0, degp, NB)

    n_chunks = -(-(-(-E // (NW * CHUNK))) // 8) * 8
    EP2 = n_chunks * CHUNK
    pad = EP2 * NW - E
    R = -(-N // (NS * 128)) * (NS * 128)
    if pad > 0 and R == N:
        R += NS * 128
    if pad > 0:
        n_dummy = R - N
        j = jnp.arange(pad, dtype=jnp.int32)
        pad_src = (j * 641) % jnp.int32(N)
        pad_dst = jnp.int32(N) + (j % jnp.int32(n_dummy))
        srcp = jnp.concatenate([src, pad_src])
        dstp = jnp.concatenate([dst, pad_dst])
    else:
        srcp, dstp = src, dst
    accp = _sc_scatter(srcp, dstp, hw, EP2, N, R)
    out = _tc_final(accp, ndst, b_conv0, W_out, b_out, NB, R)
    return out[:N]


# speedup vs baseline: 11.9970x; 1.0155x over previous
import functools

import jax
import jax.numpy as jnp
from jax import lax
from jax.experimental import pallas as pl
from jax.experimental.pallas import tpu as pltpu
from jax.experimental.pallas import tpu_sc as plsc

NC = 2
NS = 16
NW = NC * NS
L = 16
CHUNK = 128


def _sc_mesh():
    return plsc.VectorSubcoreMesh(
        core_axis_name="c", subcore_axis_name="s", num_cores=NC,
        num_subcores=NS)


@functools.partial(jax.jit, static_argnums=(2, 3))
def _sc_degrees(src, dst, E, NB):
    EP = E // NW

    @functools.partial(
        pl.kernel,
        out_type=jax.ShapeDtypeStruct((2 * NW, NB, 128), jnp.float32),
        mesh=_sc_mesh(),
        compiler_params=pltpu.CompilerParams(needs_layout_passes=False),
        scratch_types=[
            pltpu.VMEM((EP,), jnp.int32),
            pltpu.VMEM((EP,), jnp.int32),
            pltpu.VMEM((NB, 128), jnp.float32),
            pltpu.VMEM((NB, 128), jnp.float32),
        ],
    )
    def deg_kernel(src_hbm, dst_hbm, out_hbm, sbuf, dbuf, hsrc, hdst):
        cid = lax.axis_index("c")
        sid = lax.axis_index("s")
        wid = sid * NC + cid
        base = wid * EP

        zeros = jnp.zeros((L,), jnp.float32)

        def zero_body(i, _):
            r = i // 8
            j = i % 8
            hsrc[r, pl.ds(j * L, L)] = zeros
            hdst[r, pl.ds(j * L, L)] = zeros
            return 0

        lax.fori_loop(0, NB * 8, zero_body, 0)

        pltpu.sync_copy(src_hbm.at[pl.ds(base, EP)], sbuf)
        pltpu.sync_copy(dst_hbm.at[pl.ds(base, EP)], dbuf)

        ones = jnp.ones((L,), jnp.float32)
        seven = jnp.int32(7)
        mask = jnp.int32(127)

        def hist_body(i, _):
            s = sbuf[pl.ds(i * L, L)]
            d = dbuf[pl.ds(i * L, L)]
            plsc.addupdate_scatter(
                hsrc, [lax.shift_right_logical(s, seven), s & mask], ones)
            plsc.addupdate_scatter(
                hdst, [lax.shift_right_logical(d, seven), d & mask], ones)
            return 0

        lax.fori_loop(0, EP // L, hist_body, 0)

        pltpu.sync_copy(hsrc, out_hbm.at[wid])
        pltpu.sync_copy(hdst, out_hbm.at[NW + wid])

    return deg_kernel(src, dst)


def _col_scale(mat_n2_d, norm_nb_128, NB, D):
    nb = jnp.broadcast_to(norm_nb_128[:, None, :], (NB, 128, 128))
    nT = jnp.transpose(nb, (0, 2, 1))
    m3 = mat_n2_d.reshape(NB, 128, D)
    return (m3 * nT).reshape(NB * 128, D)


def _tc_dense(xp, W1, b1, W2, degp, NB):
    N2, D = xp.shape

    def body(x_ref, w1_ref, b1_ref, w2_ref, degp_ref, hw_ref, ndst_ref):
        dp = degp_ref[...]
        deg_src = jnp.sum(dp[:NW], axis=0)
        deg_dst = jnp.sum(dp[NW:], axis=0)
        nsrc = jnp.where(deg_src > 0.0,
                         lax.rsqrt(jnp.maximum(deg_src, 1.0)), 0.0)
        ndst = jnp.where(deg_dst > 0.0,
                         lax.rsqrt(jnp.maximum(deg_dst, 1.0)), 0.0)
        ndst_ref[...] = ndst
        h = jnp.dot(x_ref[...], w1_ref[...],
                    preferred_element_type=jnp.float32)
        h = jnp.maximum(h + b1_ref[...][None, :], 0.0)
        hw = jnp.dot(h, w2_ref[...], preferred_element_type=jnp.float32)
        hw_ref[...] = _col_scale(hw, nsrc, NB, D)

    return pl.pallas_call(
        body,
        out_shape=(
            jax.ShapeDtypeStruct((N2, D), jnp.float32),
            jax.ShapeDtypeStruct((NB, 128), jnp.float32),
        ),
    )(xp, W1, b1, W2, degp)


@functools.partial(jax.jit, static_argnums=(3, 4, 5))
def _sc_scatter(srcp, dstp, hw, EP2, N, R):
    D = hw.shape[1]
    n_chunks = EP2 // CHUNK
    assert n_chunks % 2 == 0
    rows_per_tile = R // NS
    zb = 64
    n_zero = rows_per_tile // zb

    @functools.partial(
        pl.kernel,
        out_type=jax.ShapeDtypeStruct((2 * R, D), jnp.float32),
        mesh=_sc_mesh(),
        scratch_types=[
            pltpu.VMEM((2, CHUNK), jnp.int32),
            pltpu.VMEM((2, CHUNK), jnp.int32),
            pltpu.VMEM((2, CHUNK), jnp.int32),
            pltpu.VMEM((2, CHUNK), jnp.int32),
            pltpu.VMEM((CHUNK, D), jnp.float32),
            pltpu.VMEM((CHUNK, D), jnp.float32),
            pltpu.VMEM((64, D), jnp.float32),
            pltpu.VMEM_SHARED((R, D), jnp.float32),
            pltpu.SemaphoreType.DMA,
            pltpu.SemaphoreType.DMA,
            pltpu.SemaphoreType.DMA,
            pltpu.SemaphoreType.DMA,
            pltpu.SemaphoreType.DMA,
            pltpu.SemaphoreType.DMA,
            pltpu.SemaphoreType.DMA,
            pltpu.SemaphoreType.DMA,
        ],
    )
    def scat_kernel(src_hbm, dst_hbm, hw_hbm, out_hbm, ixA0, ixA1, ixB0,
                    ixB1, rowsA, rowsB, zbuf, acc_sh, gsA, gsB, ssA, ssB,
                    isA0, isA1, isB0, isB1):
        cid = lax.axis_index("c")
        sid = lax.axis_index("s")
        wid = sid * NC + cid
        ebase = wid * EP2

        def load_idx(ix, sem, c):
            pltpu.async_copy(src_hbm.at[pl.ds(ebase + c * CHUNK, CHUNK)],
                             ix.at[0], sem)
            pltpu.async_copy(dst_hbm.at[pl.ds(ebase + c * CHUNK, CHUNK)],
                             ix.at[1], sem)

        def wait_idx(ix, sem):
            pltpu.make_async_copy(src_hbm.at[pl.ds(0, CHUNK)], ix.at[0],
                                  sem).wait()
            pltpu.make_async_copy(src_hbm.at[pl.ds(0, CHUNK)], ix.at[1],
                                  sem).wait()

        load_idx(ixA0, isA0, 0)
        load_idx(ixB0, isB0, 1)
        load_idx(ixA1, isA1, 2)
        load_idx(ixB1, isB1, 3)
        wait_idx(ixA0, isA0)
        pltpu.async_copy(hw_hbm.at[ixA0.at[0]], rowsA, gsA)
        wait_idx(ixB0, isB0)
        pltpu.async_copy(hw_hbm.at[ixB0.at[0]], rowsB, gsB)

        zeros = jnp.zeros((L,), jnp.float32)

        def zrow(i, _):
            r = i // (D // L)
            j = i % (D // L)
            zbuf[r, pl.ds(j * L, L)] = zeros
            return 0

        lax.fori_loop(0, zb * (D // L), zrow, 0)

        for b in range(n_zero):
            pltpu.sync_copy(
                zbuf.at[pl.ds(0, zb)],
                acc_sh.at[pl.ds(sid * rows_per_tile + b * zb, zb)])
        plsc.subcore_barrier()

        def half(c, ix_cur_A, is_cur_A, ix_nxt_A, is_nxt_A,
                 ix_cur_B, is_cur_B, ix_nxt_B, is_nxt_B):
            pltpu.make_async_copy(hw_hbm.at[ix_cur_A.at[0]], rowsA,
                                  gsA).wait()
            pltpu.async_copy(rowsA, acc_sh.at[ix_cur_A.at[1]], ssA,
                             add=True)
            pltpu.make_async_copy(hw_hbm.at[ix_cur_B.at[0]], rowsB,
                                  gsB).wait()
            pltpu.async_copy(rowsB, acc_sh.at[ix_cur_B.at[1]], ssB,
                             add=True)

            pltpu.make_async_copy(rowsA, acc_sh.at[ix_cur_A.at[1]],
                                  ssA).wait()

            @pl.when(c + 2 < n_chunks)
            def _():
                wait_idx(ix_nxt_A, is_nxt_A)
                pltpu.async_copy(hw_hbm.at[ix_nxt_A.at[0]], rowsA, gsA)

            @pl.when(c + 4 < n_chunks)
            def _():
                load_idx(ix_cur_A, is_cur_A, c + 4)

            pltpu.make_async_copy(rowsB, acc_sh.at[ix_cur_B.at[1]],
                                  ssB).wait()

            @pl.when(c + 3 < n_chunks)
            def _():
                wait_idx(ix_nxt_B, is_nxt_B)
                pltpu.async_copy(hw_hbm.at[ix_nxt_B.at[0]], rowsB, gsB)

            @pl.when(c + 5 < n_chunks)
            def _():
                load_idx(ix_cur_B, is_cur_B, c + 5)

            return 0

        def quad_body(i, _):
            c = i * 4
            half(c, ixA0, isA0, ixA1, isA1, ixB0, isB0, ixB1, isB1)
            half(c + 2, ixA1, isA1, ixA0, isA0, ixB1, isB1, ixB0, isB0)
            return 0

        lax.fori_loop(0, n_chunks // 4, quad_body, 0)
        plsc.subcore_barrier()

        pltpu.sync_copy(
            acc_sh.at[pl.ds(sid * rows_per_tile, rows_per_tile)],
            out_hbm.at[pl.ds(cid * R + sid * rows_per_tile, rows_per_tile)])

    return scat_kernel(srcp, dstp, hw)


def _tc_final(accp, ndst, b2, W3, b3, NB, R, N):
    D = accp.shape[1]
    DO = W3.shape[1]
    N2 = NB * 128

    def body(accp_ref, ndst_ref, b2_ref, w3_ref, b3_ref, out_ref):
        a = accp_ref[:N2, :] + accp_ref[R:R + N2, :]
        h2 = _col_scale(a, ndst_ref[...], NB, D) + b2_ref[...][None, :]
        h2 = jnp.maximum(h2, 0.0)
        res = (jnp.dot(h2, w3_ref[...], preferred_element_type=jnp.float32)
               + b3_ref[...][None, :])
        out_ref[...] = res[:N, :]

    return pl.pallas_call(
        body,
        out_shape=jax.ShapeDtypeStruct((N, DO), jnp.float32),
    )(accp, ndst, b2, W3, b3)


def kernel(x, edge_index, W_lin0, b_lin0, W_conv0, b_conv0, W_out, b_out):
    N, D = x.shape
    E = edge_index.shape[1]
    NB = -(-N // 128)
    N2 = NB * 128

    src = edge_index[0]
    dst = edge_index[1]

    degp = _sc_degrees(src, dst, E, NB)
    xp = jnp.pad(x, ((0, N2 - N), (0, 0))) if N2 > N else x
    hw, ndst = _tc_dense(xp, W_lin0, b_lin0, W_conv0, degp, NB)

    n_chunks = -(-(-(-E // (NW * CHUNK))) // 8) * 8
    EP2 = n_chunks * CHUNK
    pad = EP2 * NW - E
    R = -(-N // (NS * 128)) * (NS * 128)
    if pad > 0 and R == N:
        R += NS * 128
    if pad > 0:
        n_dummy = R - N
        dmask = 1
        while dmask * 2 <= n_dummy:
            dmask *= 2
        smask = 1
        while smask * 2 <= N:
            smask *= 2
        j = jnp.arange(pad, dtype=jnp.int32)
        pad_src = j & jnp.int32(smask - 1)
        pad_dst = jnp.int32(N) + (j & jnp.int32(dmask - 1))
        srcp = jnp.concatenate([src, pad_src])
        dstp = jnp.concatenate([dst, pad_dst])
    else:
        srcp, dstp = src, dst
    accp = _sc_scatter(srcp, dstp, hw, EP2, N, R)
    return _tc_final(accp, ndst, b_conv0, W_out, b_out, NB, R, N)


# speedup vs baseline: 12.2694x; 1.0227x over previous
import functools

import jax
import jax.numpy as jnp
from jax import lax
from jax.experimental import pallas as pl
from jax.experimental.pallas import tpu as pltpu
from jax.experimental.pallas import tpu_sc as plsc

NC = 2
NS = 16
NW = NC * NS
L = 16
CHUNK = 128


def _sc_mesh():
    return plsc.VectorSubcoreMesh(
        core_axis_name="c", subcore_axis_name="s", num_cores=NC,
        num_subcores=NS)


@functools.partial(jax.jit, static_argnums=(1, 2))
def _sc_degrees(edge_index, E, NB):
    EP = E // NW

    @functools.partial(
        pl.kernel,
        out_type=jax.ShapeDtypeStruct((2 * NW, NB, 128), jnp.float32),
        mesh=_sc_mesh(),
        compiler_params=pltpu.CompilerParams(needs_layout_passes=False),
        scratch_types=[
            pltpu.VMEM((EP,), jnp.int32),
            pltpu.VMEM((EP,), jnp.int32),
            pltpu.VMEM((NB, 128), jnp.float32),
            pltpu.VMEM((NB, 128), jnp.float32),
        ],
    )
    def deg_kernel(ei_hbm, out_hbm, sbuf, dbuf, hsrc, hdst):
        cid = lax.axis_index("c")
        sid = lax.axis_index("s")
        wid = sid * NC + cid
        base = wid * EP

        zeros = jnp.zeros((L,), jnp.float32)

        def zero_body(i, _):
            r = i // 8
            j = i % 8
            hsrc[r, pl.ds(j * L, L)] = zeros
            hdst[r, pl.ds(j * L, L)] = zeros
            return 0

        lax.fori_loop(0, NB * 8, zero_body, 0)

        pltpu.sync_copy(ei_hbm.at[pl.ds(base, EP)], sbuf)
        pltpu.sync_copy(ei_hbm.at[pl.ds(E + base, EP)], dbuf)

        ones = jnp.ones((L,), jnp.float32)
        seven = jnp.int32(7)
        mask = jnp.int32(127)

        def hist_body(i, _):
            s = sbuf[pl.ds(i * L, L)]
            d = dbuf[pl.ds(i * L, L)]
            plsc.addupdate_scatter(
                hsrc, [lax.shift_right_logical(s, seven), s & mask], ones)
            plsc.addupdate_scatter(
                hdst, [lax.shift_right_logical(d, seven), d & mask], ones)
            return 0

        lax.fori_loop(0, EP // L, hist_body, 0)

        pltpu.sync_copy(hsrc, out_hbm.at[wid])
        pltpu.sync_copy(hdst, out_hbm.at[NW + wid])

    return deg_kernel(edge_index.reshape(2 * E))


def _col_scale(mat_n2_d, norm_nb_128, NB, D):
    nb = jnp.broadcast_to(norm_nb_128[:, None, :], (NB, 128, 128))
    nT = jnp.transpose(nb, (0, 2, 1))
    m3 = mat_n2_d.reshape(NB, 128, D)
    return (m3 * nT).reshape(NB * 128, D)


def _tc_dense(xp, W1, b1, W2, degp, NB):
    N2, D = xp.shape

    def body(x_ref, w1_ref, b1_ref, w2_ref, degp_ref, hw_ref, ndst_ref):
        dp = degp_ref[...]
        deg_src = jnp.sum(dp[:NW], axis=0)
        deg_dst = jnp.sum(dp[NW:], axis=0)
        nsrc = jnp.where(deg_src > 0.0,
                         lax.rsqrt(jnp.maximum(deg_src, 1.0)), 0.0)
        ndst = jnp.where(deg_dst > 0.0,
                         lax.rsqrt(jnp.maximum(deg_dst, 1.0)), 0.0)
        ndst_ref[...] = ndst
        h = jnp.dot(x_ref[...], w1_ref[...],
                    preferred_element_type=jnp.float32)
        h = jnp.maximum(h + b1_ref[...][None, :], 0.0)
        hw = jnp.dot(h, w2_ref[...], preferred_element_type=jnp.float32)
        hw_ref[...] = _col_scale(hw, nsrc, NB, D)

    return pl.pallas_call(
        body,
        out_shape=(
            jax.ShapeDtypeStruct((N2, D), jnp.float32),
            jax.ShapeDtypeStruct((NB, 128), jnp.float32),
        ),
    )(xp, W1, b1, W2, degp)


@functools.partial(jax.jit, static_argnums=(3, 4, 5))
def _sc_scatter(srcp, dstp, hw, EP2, N, R):
    D = hw.shape[1]
    n_chunks = EP2 // CHUNK
    assert n_chunks % 2 == 0
    rows_per_tile = R // NS
    zb = 64
    n_zero = rows_per_tile // zb

    @functools.partial(
        pl.kernel,
        out_type=jax.ShapeDtypeStruct((2 * R, D), jnp.float32),
        mesh=_sc_mesh(),
        scratch_types=[
            pltpu.VMEM((2, CHUNK), jnp.int32),
            pltpu.VMEM((2, CHUNK), jnp.int32),
            pltpu.VMEM((2, CHUNK), jnp.int32),
            pltpu.VMEM((2, CHUNK), jnp.int32),
            pltpu.VMEM((CHUNK, D), jnp.float32),
            pltpu.VMEM((CHUNK, D), jnp.float32),
            pltpu.VMEM((64, D), jnp.float32),
            pltpu.VMEM_SHARED((R, D), jnp.float32),
            pltpu.SemaphoreType.DMA,
            pltpu.SemaphoreType.DMA,
            pltpu.SemaphoreType.DMA,
            pltpu.SemaphoreType.DMA,
            pltpu.SemaphoreType.DMA,
            pltpu.SemaphoreType.DMA,
            pltpu.SemaphoreType.DMA,
            pltpu.SemaphoreType.DMA,
        ],
    )
    def scat_kernel(src_hbm, dst_hbm, hw_hbm, out_hbm, ixA0, ixA1, ixB0,
                    ixB1, rowsA, rowsB, zbuf, acc_sh, gsA, gsB, ssA, ssB,
                    isA0, isA1, isB0, isB1):
        cid = lax.axis_index("c")
        sid = lax.axis_index("s")
        wid = sid * NC + cid
        ebase = wid * EP2

        def load_idx(ix, sem, c):
            pltpu.async_copy(src_hbm.at[pl.ds(ebase + c * CHUNK, CHUNK)],
                             ix.at[0], sem)
            pltpu.async_copy(dst_hbm.at[pl.ds(ebase + c * CHUNK, CHUNK)],
                             ix.at[1], sem)

        def wait_idx(ix, sem):
            pltpu.make_async_copy(src_hbm.at[pl.ds(0, CHUNK)], ix.at[0],
                                  sem).wait()
            pltpu.make_async_copy(src_hbm.at[pl.ds(0, CHUNK)], ix.at[1],
                                  sem).wait()

        load_idx(ixA0, isA0, 0)
        load_idx(ixB0, isB0, 1)
        load_idx(ixA1, isA1, 2)
        load_idx(ixB1, isB1, 3)
        wait_idx(ixA0, isA0)
        pltpu.async_copy(hw_hbm.at[ixA0.at[0]], rowsA, gsA)
        wait_idx(ixB0, isB0)
        pltpu.async_copy(hw_hbm.at[ixB0.at[0]], rowsB, gsB)

        zeros = jnp.zeros((L,), jnp.float32)

        def zrow(i, _):
            r = i // (D // L)
            j = i % (D // L)
            zbuf[r, pl.ds(j * L, L)] = zeros
            return 0

        lax.fori_loop(0, zb * (D // L), zrow, 0)

        for b in range(n_zero):
            pltpu.sync_copy(
                zbuf.at[pl.ds(0, zb)],
                acc_sh.at[pl.ds(sid * rows_per_tile + b * zb, zb)])
        plsc.subcore_barrier()

        def half(c, ix_cur_A, is_cur_A, ix_nxt_A, is_nxt_A,
                 ix_cur_B, is_cur_B, ix_nxt_B, is_nxt_B):
            pltpu.make_async_copy(hw_hbm.at[ix_cur_A.at[0]], rowsA,
                                  gsA).wait()
            pltpu.async_copy(rowsA, acc_sh.at[ix_cur_A.at[1]], ssA,
                             add=True)
            pltpu.make_async_copy(hw_hbm.at[ix_cur_B.at[0]], rowsB,
                                  gsB).wait()
            pltpu.async_copy(rowsB, acc_sh.at[ix_cur_B.at[1]], ssB,
                             add=True)

            pltpu.make_async_copy(rowsA, acc_sh.at[ix_cur_A.at[1]],
                                  ssA).wait()

            @pl.when(c + 2 < n_chunks)
            def _():
                wait_idx(ix_nxt_A, is_nxt_A)
                pltpu.async_copy(hw_hbm.at[ix_nxt_A.at[0]], rowsA, gsA)

            @pl.when(c + 4 < n_chunks)
            def _():
                load_idx(ix_cur_A, is_cur_A, c + 4)

            pltpu.make_async_copy(rowsB, acc_sh.at[ix_cur_B.at[1]],
                                  ssB).wait()

            @pl.when(c + 3 < n_chunks)
            def _():
                wait_idx(ix_nxt_B, is_nxt_B)
                pltpu.async_copy(hw_hbm.at[ix_nxt_B.at[0]], rowsB, gsB)

            @pl.when(c + 5 < n_chunks)
            def _():
                load_idx(ix_cur_B, is_cur_B, c + 5)

            return 0

        def quad_body(i, _):
            c = i * 4
            half(c, ixA0, isA0, ixA1, isA1, ixB0, isB0, ixB1, isB1)
            half(c + 2, ixA1, isA1, ixA0, isA0, ixB1, isB1, ixB0, isB0)
            return 0

        lax.fori_loop(0, n_chunks // 4, quad_body, 0)
        plsc.subcore_barrier()

        pltpu.sync_copy(
            acc_sh.at[pl.ds(sid * rows_per_tile, rows_per_tile)],
            out_hbm.at[pl.ds(cid * R + sid * rows_per_tile, rows_per_tile)])

    return scat_kernel(srcp, dstp, hw)


def _tc_final(accp, ndst, b2, W3, b3, NB, R, N):
    D = accp.shape[1]
    DO = W3.shape[1]
    N2 = NB * 128

    def body(accp_ref, ndst_ref, b2_ref, w3_ref, b3_ref, out_ref):
        a = accp_ref[:N2, :] + accp_ref[R:R + N2, :]
        h2 = _col_scale(a, ndst_ref[...], NB, D) + b2_ref[...][None, :]
        h2 = jnp.maximum(h2, 0.0)
        res = (jnp.dot(h2, w3_ref[...], preferred_element_type=jnp.float32)
               + b3_ref[...][None, :])
        out_ref[...] = res[:N, :]

    return pl.pallas_call(
        body,
        out_shape=jax.ShapeDtypeStruct((N, DO), jnp.float32),
    )(accp, ndst, b2, W3, b3)


def kernel(x, edge_index, W_lin0, b_lin0, W_conv0, b_conv0, W_out, b_out):
    N, D = x.shape
    E = edge_index.shape[1]
    NB = -(-N // 128)
    N2 = NB * 128

    src = edge_index[0]
    dst = edge_index[1]

    degp = _sc_degrees(edge_index, E, NB)
    xp = jnp.pad(x, ((0, N2 - N), (0, 0))) if N2 > N else x
    hw, ndst = _tc_dense(xp, W_lin0, b_lin0, W_conv0, degp, NB)

    n_chunks = -(-(-(-E // (NW * CHUNK))) // 8) * 8
    EP2 = n_chunks * CHUNK
    pad = EP2 * NW - E
    R = -(-N // (NS * 128)) * (NS * 128)
    if pad > 0 and R == N:
        R += NS * 128
    if pad > 0:
        n_dummy = R - N
        dmask = 1
        while dmask * 2 <= n_dummy:
            dmask *= 2
        smask = 1
        while smask * 2 <= N:
            smask *= 2
        j = jnp.arange(pad, dtype=jnp.int32)
        pad_src = j & jnp.int32(smask - 1)
        pad_dst = jnp.int32(N) + (j & jnp.int32(dmask - 1))
        srcp = jnp.concatenate([src, pad_src])
        dstp = jnp.concatenate([dst, pad_dst])
    else:
        srcp, dstp = src, dst
    accp = _sc_scatter(srcp, dstp, hw, EP2, N, R)
    return _tc_final(accp, ndst, b_conv0, W_out, b_out, NB, R, N)


# speedup vs baseline: 12.5314x; 1.0214x over previous
import functools

import jax
import jax.numpy as jnp
from jax import lax
from jax.experimental import pallas as pl
from jax.experimental.pallas import tpu as pltpu
from jax.experimental.pallas import tpu_sc as plsc

NC = 2
NS = 16
NW = NC * NS
L = 16
CHUNK = 128


def _sc_mesh():
    return plsc.VectorSubcoreMesh(
        core_axis_name="c", subcore_axis_name="s", num_cores=NC,
        num_subcores=NS)


@functools.partial(jax.jit, static_argnums=(1, 2))
def _sc_degrees(edge_index, E, NB):
    EP = E // NW
    EP_AL = -(-(EP + 112) // 128) * 128
    assert (NW - 1) * EP - ((NW - 1) * EP) % 128 + EP_AL <= E

    @functools.partial(
        pl.kernel,
        out_type=jax.ShapeDtypeStruct((2 * NW, NB, 128), jnp.float32),
        mesh=_sc_mesh(),
        compiler_params=pltpu.CompilerParams(needs_layout_passes=False),
        scratch_types=[
            pltpu.VMEM((2, EP_AL), jnp.int32),
            pltpu.VMEM((NB, 128), jnp.float32),
            pltpu.VMEM((NB, 128), jnp.float32),
        ],
    )
    def deg_kernel(ei_hbm, out_hbm, sdbuf, hsrc, hdst):
        cid = lax.axis_index("c")
        sid = lax.axis_index("s")
        wid = sid * NC + cid
        base = wid * EP
        base_al = (base // 128) * 128
        off = base - base_al

        zeros = jnp.zeros((L,), jnp.float32)

        def zero_body(i, _):
            r = i // 8
            j = i % 8
            hsrc[r, pl.ds(j * L, L)] = zeros
            hdst[r, pl.ds(j * L, L)] = zeros
            return 0

        lax.fori_loop(0, NB * 8, zero_body, 0)

        pltpu.sync_copy(
            ei_hbm.at[pl.ds(0, 2), pl.ds(base_al, EP_AL)], sdbuf)

        ones = jnp.ones((L,), jnp.float32)
        seven = jnp.int32(7)
        mask = jnp.int32(127)

        def hist_body(i, _):
            s = sdbuf[0, pl.ds(off + i * L, L)]
            d = sdbuf[1, pl.ds(off + i * L, L)]
            plsc.addupdate_scatter(
                hsrc, [lax.shift_right_logical(s, seven), s & mask], ones)
            plsc.addupdate_scatter(
                hdst, [lax.shift_right_logical(d, seven), d & mask], ones)
            return 0

        lax.fori_loop(0, EP // L, hist_body, 0)

        pltpu.sync_copy(hsrc, out_hbm.at[wid])
        pltpu.sync_copy(hdst, out_hbm.at[NW + wid])

    return deg_kernel(edge_index)


def _col_scale(mat_n2_d, norm_nb_128, NB, D):
    nb = jnp.broadcast_to(norm_nb_128[:, None, :], (NB, 128, 128))
    nT = jnp.transpose(nb, (0, 2, 1))
    m3 = mat_n2_d.reshape(NB, 128, D)
    return (m3 * nT).reshape(NB * 128, D)


def _tc_matmul(xp, W1, b1, W2):
    N2, D = xp.shape

    def body(x_ref, w1_ref, b1_ref, w2_ref, h2_ref):
        h = jnp.dot(x_ref[...], w1_ref[...],
                    preferred_element_type=jnp.float32)
        h = jnp.maximum(h + b1_ref[...][None, :], 0.0)
        h2_ref[...] = jnp.dot(h, w2_ref[...],
                              preferred_element_type=jnp.float32)

    return pl.pallas_call(
        body,
        out_shape=jax.ShapeDtypeStruct((N2, D), jnp.float32),
    )(xp, W1, b1, W2)


def _tc_scale(h2, degp, NB):
    N2, D = h2.shape

    def body(h2_ref, degp_ref, hw_ref, ndst_ref):
        dp = degp_ref[...]
        deg_src = jnp.sum(dp[:NW], axis=0)
        deg_dst = jnp.sum(dp[NW:], axis=0)
        nsrc = jnp.where(deg_src > 0.0,
                         lax.rsqrt(jnp.maximum(deg_src, 1.0)), 0.0)
        ndst = jnp.where(deg_dst > 0.0,
                         lax.rsqrt(jnp.maximum(deg_dst, 1.0)), 0.0)
        ndst_ref[...] = ndst
        hw_ref[...] = _col_scale(h2_ref[...], nsrc, NB, D)

    return pl.pallas_call(
        body,
        out_shape=(
            jax.ShapeDtypeStruct((N2, D), jnp.float32),
            jax.ShapeDtypeStruct((NB, 128), jnp.float32),
        ),
    )(h2, degp)


@functools.partial(jax.jit, static_argnums=(3, 4, 5))
def _sc_scatter(srcp, dstp, hw, EP2, N, R):
    D = hw.shape[1]
    n_chunks = EP2 // CHUNK
    assert n_chunks % 2 == 0
    rows_per_tile = R // NS
    zb = 64
    n_zero = rows_per_tile // zb

    @functools.partial(
        pl.kernel,
        out_type=jax.ShapeDtypeStruct((2 * R, D), jnp.float32),
        mesh=_sc_mesh(),
        scratch_types=[
            pltpu.VMEM((2, CHUNK), jnp.int32),
            pltpu.VMEM((2, CHUNK), jnp.int32),
            pltpu.VMEM((2, CHUNK), jnp.int32),
            pltpu.VMEM((2, CHUNK), jnp.int32),
            pltpu.VMEM((CHUNK, D), jnp.float32),
            pltpu.VMEM((CHUNK, D), jnp.float32),
            pltpu.VMEM((64, D), jnp.float32),
            pltpu.VMEM_SHARED((R, D), jnp.float32),
            pltpu.SemaphoreType.DMA,
            pltpu.SemaphoreType.DMA,
            pltpu.SemaphoreType.DMA,
            pltpu.SemaphoreType.DMA,
            pltpu.SemaphoreType.DMA,
            pltpu.SemaphoreType.DMA,
            pltpu.SemaphoreType.DMA,
            pltpu.SemaphoreType.DMA,
        ],
    )
    def scat_kernel(src_hbm, dst_hbm, hw_hbm, out_hbm, ixA0, ixA1, ixB0,
                    ixB1, rowsA, rowsB, zbuf, acc_sh, gsA, gsB, ssA, ssB,
                    isA0, isA1, isB0, isB1):
        cid = lax.axis_index("c")
        sid = lax.axis_index("s")
        wid = sid * NC + cid
        ebase = wid * EP2

        def load_idx(ix, sem, c):
            pltpu.async_copy(src_hbm.at[pl.ds(ebase + c * CHUNK, CHUNK)],
                             ix.at[0], sem)
            pltpu.async_copy(dst_hbm.at[pl.ds(ebase + c * CHUNK, CHUNK)],
                             ix.at[1], sem)

        def wait_idx(ix, sem):
            pltpu.make_async_copy(src_hbm.at[pl.ds(0, CHUNK)], ix.at[0],
                                  sem).wait()
            pltpu.make_async_copy(src_hbm.at[pl.ds(0, CHUNK)], ix.at[1],
                                  sem).wait()

        load_idx(ixA0, isA0, 0)
        load_idx(ixB0, isB0, 1)
        load_idx(ixA1, isA1, 2)
        load_idx(ixB1, isB1, 3)
        wait_idx(ixA0, isA0)
        pltpu.async_copy(hw_hbm.at[ixA0.at[0]], rowsA, gsA)
        wait_idx(ixB0, isB0)
        pltpu.async_copy(hw_hbm.at[ixB0.at[0]], rowsB, gsB)

        zeros = jnp.zeros((L,), jnp.float32)

        def zrow(i, _):
            r = i // (D // L)
            j = i % (D // L)
            zbuf[r, pl.ds(j * L, L)] = zeros
            return 0

        lax.fori_loop(0, zb * (D // L), zrow, 0)

        for b in range(n_zero):
            pltpu.sync_copy(
                zbuf.at[pl.ds(0, zb)],
                acc_sh.at[pl.ds(sid * rows_per_tile + b * zb, zb)])
        plsc.subcore_barrier()

        def half(c, ix_cur_A, is_cur_A, ix_nxt_A, is_nxt_A,
                 ix_cur_B, is_cur_B, ix_nxt_B, is_nxt_B):
            pltpu.make_async_copy(hw_hbm.at[ix_cur_A.at[0]], rowsA,
                                  gsA).wait()
            pltpu.async_copy(rowsA, acc_sh.at[ix_cur_A.at[1]], ssA,
                             add=True)
            pltpu.make_async_copy(hw_hbm.at[ix_cur_B.at[0]], rowsB,
                                  gsB).wait()
            pltpu.async_copy(rowsB, acc_sh.at[ix_cur_B.at[1]], ssB,
                             add=True)

            pltpu.make_async_copy(rowsA, acc_sh.at[ix_cur_A.at[1]],
                                  ssA).wait()

            @pl.when(c + 2 < n_chunks)
            def _():
                wait_idx(ix_nxt_A, is_nxt_A)
                pltpu.async_copy(hw_hbm.at[ix_nxt_A.at[0]], rowsA, gsA)

            @pl.when(c + 4 < n_chunks)
            def _():
                load_idx(ix_cur_A, is_cur_A, c + 4)

            pltpu.make_async_copy(rowsB, acc_sh.at[ix_cur_B.at[1]],
                                  ssB).wait()

            @pl.when(c + 3 < n_chunks)
            def _():
                wait_idx(ix_nxt_B, is_nxt_B)
                pltpu.async_copy(hw_hbm.at[ix_nxt_B.at[0]], rowsB, gsB)

            @pl.when(c + 5 < n_chunks)
            def _():
                load_idx(ix_cur_B, is_cur_B, c + 5)

            return 0

        def quad_body(i, _):
            c = i * 4
            half(c, ixA0, isA0, ixA1, isA1, ixB0, isB0, ixB1, isB1)
            half(c + 2, ixA1, isA1, ixA0, isA0, ixB1, isB1, ixB0, isB0)
            return 0

        lax.fori_loop(0, n_chunks // 4, quad_body, 0)
        plsc.subcore_barrier()

        pltpu.sync_copy(
            acc_sh.at[pl.ds(sid * rows_per_tile, rows_per_tile)],
            out_hbm.at[pl.ds(cid * R + sid * rows_per_tile, rows_per_tile)])

    return scat_kernel(srcp, dstp, hw)


def _tc_final(accp, ndst, b2, W3, b3, NB, R, N):
    D = accp.shape[1]
    DO = W3.shape[1]
    N2 = NB * 128

    def body(accp_ref, ndst_ref, b2_ref, w3_ref, b3_ref, out_ref):
        a = accp_ref[:N2, :] + accp_ref[R:R + N2, :]
        h2 = _col_scale(a, ndst_ref[...], NB, D) + b2_ref[...][None, :]
        h2 = jnp.maximum(h2, 0.0)
        res = lax.dot_general(w3_ref[...], h2,
                              (((0,), (1,)), ((), ())),
                              preferred_element_type=jnp.float32)
        out_ref[...] = res + b3_ref[...][:, None]

    return pl.pallas_call(
        body,
        out_shape=jax.ShapeDtypeStruct((DO, N2), jnp.float32),
    )(accp, ndst, b2, W3, b3)


def kernel(x, edge_index, W_lin0, b_lin0, W_conv0, b_conv0, W_out, b_out):
    N, D = x.shape
    E = edge_index.shape[1]
    NB = -(-N // 128)
    N2 = NB * 128

    src = edge_index[0]
    dst = edge_index[1]

    degp = _sc_degrees(edge_index, E, NB)
    xp = jnp.pad(x, ((0, N2 - N), (0, 0))) if N2 > N else x
    h2 = _tc_matmul(xp, W_lin0, b_lin0, W_conv0)
    hw, ndst = _tc_scale(h2, degp, NB)

    n_chunks = -(-(-(-E // (NW * CHUNK))) // 8) * 8
    EP2 = n_chunks * CHUNK
    pad = EP2 * NW - E
    R = -(-N // (NS * 128)) * (NS * 128)
    if pad > 0 and R == N:
        R += NS * 128
    if pad > 0:
        n_dummy = R - N
        dmask = 1
        while dmask * 2 <= n_dummy:
            dmask *= 2
        smask = 1
        while smask * 2 <= N:
            smask *= 2
        j = jnp.arange(pad, dtype=jnp.int32)
        pad_src = j & jnp.int32(smask - 1)
        pad_dst = jnp.int32(N) + (j & jnp.int32(dmask - 1))
        srcp = jnp.concatenate([src, pad_src])
        dstp = jnp.concatenate([dst, pad_dst])
    else:
        srcp, dstp = src, dst
    accp = _sc_scatter(srcp, dstp, hw, EP2, N, R)
    outT = _tc_final(accp, ndst, b_conv0, W_out, b_out, NB, R, N)
    return outT.T[:N]


# speedup vs baseline: 13.1992x; 1.0533x over previous
import functools

import jax
import jax.numpy as jnp
from jax import lax
from jax.experimental import pallas as pl
from jax.experimental.pallas import tpu as pltpu
from jax.experimental.pallas import tpu_sc as plsc

NC = 2
NS = 16
NW = NC * NS
L = 16
CHUNK = 128


def _sc_mesh():
    return plsc.VectorSubcoreMesh(
        core_axis_name="c", subcore_axis_name="s", num_cores=NC,
        num_subcores=NS)


@functools.partial(jax.jit, static_argnums=(1, 2))
def _sc_degrees(edge_index, E, NB):
    EP = E // NW
    EP_AL = -(-(EP + 112) // 128) * 128
    assert (NW - 1) * EP - ((NW - 1) * EP) % 128 + EP_AL <= E

    @functools.partial(
        pl.kernel,
        out_type=jax.ShapeDtypeStruct((2 * NW, NB, 128), jnp.float32),
        mesh=_sc_mesh(),
        compiler_params=pltpu.CompilerParams(needs_layout_passes=False),
        scratch_types=[
            pltpu.VMEM((2, EP_AL), jnp.int32),
            pltpu.VMEM((NB, 128), jnp.float32),
            pltpu.VMEM((NB, 128), jnp.float32),
        ],
    )
    def deg_kernel(ei_hbm, out_hbm, sdbuf, hsrc, hdst):
        cid = lax.axis_index("c")
        sid = lax.axis_index("s")
        wid = sid * NC + cid
        base = wid * EP
        base_al = (base // 128) * 128
        off = base - base_al

        zeros = jnp.zeros((L,), jnp.float32)

        def zero_body(i, _):
            r = i // 8
            j = i % 8
            hsrc[r, pl.ds(j * L, L)] = zeros
            hdst[r, pl.ds(j * L, L)] = zeros
            return 0

        lax.fori_loop(0, NB * 8, zero_body, 0)

        pltpu.sync_copy(
            ei_hbm.at[pl.ds(0, 2), pl.ds(base_al, EP_AL)], sdbuf)

        ones = jnp.ones((L,), jnp.float32)
        seven = jnp.int32(7)
        mask = jnp.int32(127)

        def hist_body(i, _):
            s = sdbuf[0, pl.ds(off + i * L, L)]
            d = sdbuf[1, pl.ds(off + i * L, L)]
            plsc.addupdate_scatter(
                hsrc, [lax.shift_right_logical(s, seven), s & mask], ones)
            plsc.addupdate_scatter(
                hdst, [lax.shift_right_logical(d, seven), d & mask], ones)
            return 0

        lax.fori_loop(0, EP // L, hist_body, 0)

        pltpu.sync_copy(hsrc, out_hbm.at[wid])
        pltpu.sync_copy(hdst, out_hbm.at[NW + wid])

    return deg_kernel(edge_index)


def _col_scale(mat_n2_d, norm_nb_128, NB, D):
    nb = jnp.broadcast_to(norm_nb_128[:, None, :], (NB, 128, 128))
    nT = jnp.transpose(nb, (0, 2, 1))
    m3 = mat_n2_d.reshape(NB, 128, D)
    return (m3 * nT).reshape(NB * 128, D)


def _tc_matmul(xp, W1, b1, W2, edge_index, X, N, smask, dmask):
    N2, D = xp.shape
    E = edge_index.shape[1]
    PAD = X - E

    def body(x_ref, w1_ref, b1_ref, w2_ref, ei_ref, h2_ref, eo_ref):
        h = jnp.dot(x_ref[...], w1_ref[...],
                    preferred_element_type=jnp.float32)
        h = jnp.maximum(h + b1_ref[...][None, :], 0.0)
        h2_ref[...] = jnp.dot(h, w2_ref[...],
                              preferred_element_type=jnp.float32)
        eo_ref[0, pl.ds(0, E)] = ei_ref[0, :]
        eo_ref[1, pl.ds(0, E)] = ei_ref[1, :]
        j = lax.broadcasted_iota(jnp.int32, (PAD,), 0)
        eo_ref[0, pl.ds(E, PAD)] = j & jnp.int32(smask - 1)
        eo_ref[1, pl.ds(E, PAD)] = jnp.int32(N) + (j & jnp.int32(dmask - 1))

    return pl.pallas_call(
        body,
        out_shape=(
            jax.ShapeDtypeStruct((N2, D), jnp.float32),
            jax.ShapeDtypeStruct((2, X), jnp.int32),
        ),
    )(xp, W1, b1, W2, edge_index)


def _tc_scale(h2, degp, NB):
    N2, D = h2.shape

    def body(h2_ref, degp_ref, hw_ref, ndst_ref):
        dp = degp_ref[...]
        deg_src = jnp.sum(dp[:NW], axis=0)
        deg_dst = jnp.sum(dp[NW:], axis=0)
        nsrc = jnp.where(deg_src > 0.0,
                         lax.rsqrt(jnp.maximum(deg_src, 1.0)), 0.0)
        ndst = jnp.where(deg_dst > 0.0,
                         lax.rsqrt(jnp.maximum(deg_dst, 1.0)), 0.0)
        ndst_ref[...] = ndst
        hw_ref[...] = _col_scale(h2_ref[...], nsrc, NB, D)

    return pl.pallas_call(
        body,
        out_shape=(
            jax.ShapeDtypeStruct((N2, D), jnp.float32),
            jax.ShapeDtypeStruct((NB, 128), jnp.float32),
        ),
    )(h2, degp)


@functools.partial(jax.jit, static_argnums=(2, 3, 4))
def _sc_scatter(eo, hw, EP2, N, R):
    D = hw.shape[1]
    n_chunks = EP2 // CHUNK
    assert n_chunks % 2 == 0
    rows_per_tile = R // NS
    zb = 64
    n_zero = rows_per_tile // zb

    @functools.partial(
        pl.kernel,
        out_type=jax.ShapeDtypeStruct((2 * R, D), jnp.float32),
        mesh=_sc_mesh(),
        scratch_types=[
            pltpu.VMEM((2, CHUNK), jnp.int32),
            pltpu.VMEM((2, CHUNK), jnp.int32),
            pltpu.VMEM((2, CHUNK), jnp.int32),
            pltpu.VMEM((2, CHUNK), jnp.int32),
            pltpu.VMEM((CHUNK, D), jnp.float32),
            pltpu.VMEM((CHUNK, D), jnp.float32),
            pltpu.VMEM((64, D), jnp.float32),
            pltpu.VMEM_SHARED((R, D), jnp.float32),
            pltpu.SemaphoreType.DMA,
            pltpu.SemaphoreType.DMA,
            pltpu.SemaphoreType.DMA,
            pltpu.SemaphoreType.DMA,
            pltpu.SemaphoreType.DMA,
            pltpu.SemaphoreType.DMA,
            pltpu.SemaphoreType.DMA,
            pltpu.SemaphoreType.DMA,
        ],
    )
    def scat_kernel(eo_hbm, hw_hbm, out_hbm, ixA0, ixA1, ixB0,
                    ixB1, rowsA, rowsB, zbuf, acc_sh, gsA, gsB, ssA, ssB,
                    isA0, isA1, isB0, isB1):
        cid = lax.axis_index("c")
        sid = lax.axis_index("s")
        wid = sid * NC + cid
        ebase = wid * EP2

        def load_idx(ix, sem, c):
            pltpu.async_copy(
                eo_hbm.at[pl.ds(0, 2), pl.ds(ebase + c * CHUNK, CHUNK)],
                ix, sem)

        def wait_idx(ix, sem):
            pltpu.make_async_copy(
                eo_hbm.at[pl.ds(0, 2), pl.ds(0, CHUNK)], ix, sem).wait()

        load_idx(ixA0, isA0, 0)
        load_idx(ixB0, isB0, 1)
        load_idx(ixA1, isA1, 2)
        load_idx(ixB1, isB1, 3)
        wait_idx(ixA0, isA0)
        pltpu.async_copy(hw_hbm.at[ixA0.at[0]], rowsA, gsA)
        wait_idx(ixB0, isB0)
        pltpu.async_copy(hw_hbm.at[ixB0.at[0]], rowsB, gsB)

        zeros = jnp.zeros((L,), jnp.float32)

        def zrow(i, _):
            r = i // (D // L)
            j = i % (D // L)
            zbuf[r, pl.ds(j * L, L)] = zeros
            return 0

        lax.fori_loop(0, zb * (D // L), zrow, 0)

        for b in range(n_zero):
            pltpu.sync_copy(
                zbuf.at[pl.ds(0, zb)],
                acc_sh.at[pl.ds(sid * rows_per_tile + b * zb, zb)])
        plsc.subcore_barrier()

        def half(c, ix_cur_A, is_cur_A, ix_nxt_A, is_nxt_A,
                 ix_cur_B, is_cur_B, ix_nxt_B, is_nxt_B):
            pltpu.make_async_copy(hw_hbm.at[ix_cur_A.at[0]], rowsA,
                                  gsA).wait()
            pltpu.async_copy(rowsA, acc_sh.at[ix_cur_A.at[1]], ssA,
                             add=True)
            pltpu.make_async_copy(hw_hbm.at[ix_cur_B.at[0]], rowsB,
                                  gsB).wait()
            pltpu.async_copy(rowsB, acc_sh.at[ix_cur_B.at[1]], ssB,
                             add=True)

            pltpu.make_async_copy(rowsA, acc_sh.at[ix_cur_A.at[1]],
                                  ssA).wait()

            @pl.when(c + 2 < n_chunks)
            def _():
                wait_idx(ix_nxt_A, is_nxt_A)
                pltpu.async_copy(hw_hbm.at[ix_nxt_A.at[0]], rowsA, gsA)

            @pl.when(c + 4 < n_chunks)
            def _():
                load_idx(ix_cur_A, is_cur_A, c + 4)

            pltpu.make_async_copy(rowsB, acc_sh.at[ix_cur_B.at[1]],
                                  ssB).wait()

            @pl.when(c + 3 < n_chunks)
            def _():
                wait_idx(ix_nxt_B, is_nxt_B)
                pltpu.async_copy(hw_hbm.at[ix_nxt_B.at[0]], rowsB, gsB)

            @pl.when(c + 5 < n_chunks)
            def _():
                load_idx(ix_cur_B, is_cur_B, c + 5)

            return 0

        def quad_body(i, _):
            c = i * 4
            half(c, ixA0, isA0, ixA1, isA1, ixB0, isB0, ixB1, isB1)
            half(c + 2, ixA1, isA1, ixA0, isA0, ixB1, isB1, ixB0, isB0)
            return 0

        lax.fori_loop(0, n_chunks // 4, quad_body, 0)
        plsc.subcore_barrier()

        pltpu.sync_copy(
            acc_sh.at[pl.ds(sid * rows_per_tile, rows_per_tile)],
            out_hbm.at[pl.ds(cid * R + sid * rows_per_tile, rows_per_tile)])

    return scat_kernel(eo, hw)


def _tc_final(accp, ndst, b2, W3, b3, NB, R, N):
    D = accp.shape[1]
    DO = W3.shape[1]
    N2 = NB * 128

    def body(accp_ref, ndst_ref, b2_ref, w3_ref, b3_ref, out_ref):
        a = accp_ref[:N2, :] + accp_ref[R:R + N2, :]
        h2 = _col_scale(a, ndst_ref[...], NB, D) + b2_ref[...][None, :]
        h2 = jnp.maximum(h2, 0.0)
        res = lax.dot_general(w3_ref[...], h2,
                              (((0,), (1,)), ((), ())),
                              preferred_element_type=jnp.float32)
        out_ref[...] = res + b3_ref[...][:, None]

    return pl.pallas_call(
        body,
        out_shape=jax.ShapeDtypeStruct((DO, N2), jnp.float32),
    )(accp, ndst, b2, W3, b3)


def kernel(x, edge_index, W_lin0, b_lin0, W_conv0, b_conv0, W_out, b_out):
    N, D = x.shape
    E = edge_index.shape[1]
    NB = -(-N // 128)
    N2 = NB * 128

    degp = _sc_degrees(edge_index, E, NB)
    xp = jnp.pad(x, ((0, N2 - N), (0, 0))) if N2 > N else x

    n_chunks = -(-(-(-E // (NW * CHUNK))) // 8) * 8
    EP2 = n_chunks * CHUNK
    X = EP2 * NW
    R = -(-N // (NS * 128)) * (NS * 128)
    if X > E and R == N:
        R += NS * 128
    n_dummy = max(R - N, 1)
    dmask = 1
    while dmask * 2 <= n_dummy:
        dmask *= 2
    smask = 1
    while smask * 2 <= N:
        smask *= 2

    h2, eo = _tc_matmul(xp, W_lin0, b_lin0, W_conv0, edge_index, X, N,
                        smask, dmask)
    hw, ndst = _tc_scale(h2, degp, NB)
    accp = _sc_scatter(eo, hw, EP2, N, R)
    outT = _tc_final(accp, ndst, b_conv0, W_out, b_out, NB, R, N)
    return outT.T[:N]
